# baseline (speedup 1.0000x reference)
import functools

import jax
import jax.numpy as jnp
from jax.experimental import pallas as pl
from jax.experimental.pallas import tpu as pltpu

_T = 128
_R = 8


def _rot_down(a, k):
    return pltpu.roll(a, k % _R, 0)


def _a_body(x_ref, id_ref, l_ref, c_ref, sid_ref):
    i = pl.program_id(0)
    T, D = x_ref.shape
    nv = T // _R
    neg = jnp.float32(-jnp.inf)

    first_id = id_ref[0, 0]
    last_id = id_ref[T - 1, 0]
    single = first_id == last_id

    c_id_prev = jnp.where(i == 0, -1, sid_ref[0])
    c_vec_prev = c_ref[0:1, :]
    l_row = jnp.where(first_id == c_id_prev, c_vec_prev,
                      jnp.full_like(c_vec_prev, neg))
    l_ref[0, 0:1, :] = l_row

    acc = None
    for v in range(nv):
        idb = jnp.broadcast_to(id_ref[pl.ds(v * _R, _R), :], (_R, D))
        xv = x_ref[pl.ds(v * _R, _R), :]
        contrib = jnp.where(idb == last_id, xv, neg)
        acc = contrib if acc is None else jnp.maximum(acc, contrib)
    tailmax = jnp.max(acc, axis=0, keepdims=True)

    c_ref[0:1, :] = jnp.maximum(
        tailmax, jnp.where(single, l_row, jnp.full_like(l_row, neg)))
    sid_ref[0] = last_id


def _b_body(x_ref, id_ref, l_ref, out_ref, r_ref, sid_ref):
    i = pl.program_id(0)
    T, D = x_ref.shape
    nv = T // _R
    neg = jnp.float32(-jnp.inf)

    first_id = id_ref[0, 0]
    last_id = id_ref[T - 1, 0]
    single = first_id == last_id

    r_id_prev = jnp.where(i == 0, -1, sid_ref[0])
    r_vec_prev = r_ref[0:1, :]

    idb = []
    f = []
    b = []
    for v in range(nv):
        idb.append(jnp.broadcast_to(id_ref[pl.ds(v * _R, _R), :], (_R, D)))
        xv = x_ref[pl.ds(v * _R, _R), :]
        f.append(xv)
        b.append(xv)

    for k in (1, 2, 4):
        nf, nb = [], []
        for v in range(nv):
            id_d = _rot_down(idb[v], k)
            id_u = _rot_down(idb[v], -k)
            nf.append(jnp.where(idb[v] == id_d,
                                jnp.maximum(f[v], _rot_down(f[v], k)), f[v]))
            nb.append(jnp.where(idb[v] == id_u,
                                jnp.maximum(b[v], _rot_down(b[v], -k)), b[v]))
        f, b = nf, nb

    gf = [jnp.broadcast_to(f[v][_R - 1:_R, :], (_R, D)) for v in range(nv)]
    id_l = [jnp.broadcast_to(idb[v][_R - 1:_R, :], (_R, D)) for v in range(nv)]
    gb = [jnp.broadcast_to(b[v][0:1, :], (_R, D)) for v in range(nv)]
    id_f = [jnp.broadcast_to(idb[v][0:1, :], (_R, D)) for v in range(nv)]

    off = 1
    while off < nv:
        ngf, ngb = [], []
        for v in range(nv):
            d = (v - off) % nv
            u = (v + off) % nv
            ngf.append(jnp.where(id_l[v] == id_l[d],
                                 jnp.maximum(gf[v], gf[d]), gf[v]))
            ngb.append(jnp.where(id_f[v] == id_f[u],
                                 jnp.maximum(gb[v], gb[u]), gb[v]))
        gf, gb = ngf, ngb
        off *= 2

    nf, nb = [], []
    for v in range(nv):
        d = (v - 1) % nv
        u = (v + 1) % nv
        nf.append(jnp.where(idb[v] == id_l[d],
                            jnp.maximum(f[v], gf[d]), f[v]))
        nb.append(jnp.where(idb[v] == id_f[u],
                            jnp.maximum(b[v], gb[u]), b[v]))
    f, b = nf, nb

    l_row = l_ref[0, 0:1, :]
    l_bc = jnp.broadcast_to(l_row, (_R, D))
    r_bc = jnp.broadcast_to(r_vec_prev, (_R, D))
    for v in range(nv):
        m = jnp.maximum(f[v], b[v])
        m = jnp.where(idb[v] == first_id, jnp.maximum(m, l_bc), m)
        m = jnp.where(idb[v] == r_id_prev, jnp.maximum(m, r_bc), m)
        out_ref[pl.ds(v * _R, _R), :] = m

    r_new = jnp.maximum(
        b[0][0:1, :],
        jnp.where(jnp.logical_and(single, first_id == r_id_prev),
                  r_vec_prev, jnp.full_like(r_vec_prev, neg)))
    r_ref[0:1, :] = r_new
    sid_ref[0] = first_id


def kernel(x_sp, vertex_id):
    E, D = x_sp.shape
    T = _T
    nt = E // T
    idcol = vertex_id.reshape(E, 1)

    l_table = pl.pallas_call(
        _a_body,
        grid=(nt,),
        in_specs=[
            pl.BlockSpec((T, D), lambda i: (i, 0)),
            pl.BlockSpec((T, 1), lambda i: (i, 0)),
        ],
        out_specs=pl.BlockSpec((1, 1, D), lambda i: (i, 0, 0)),
        out_shape=jax.ShapeDtypeStruct((nt, 1, D), jnp.float32),
        scratch_shapes=[
            pltpu.VMEM((1, D), jnp.float32),
            pltpu.SMEM((1,), jnp.int32),
        ],
    )(x_sp, idcol)

    return pl.pallas_call(
        _b_body,
        grid=(nt,),
        in_specs=[
            pl.BlockSpec((T, D), lambda i: (nt - 1 - i, 0)),
            pl.BlockSpec((T, 1), lambda i: (nt - 1 - i, 0)),
            pl.BlockSpec((1, 1, D), lambda i: (nt - 1 - i, 0, 0)),
        ],
        out_specs=pl.BlockSpec((T, D), lambda i: (nt - 1 - i, 0)),
        out_shape=jax.ShapeDtypeStruct((E, D), jnp.float32),
        scratch_shapes=[
            pltpu.VMEM((1, D), jnp.float32),
            pltpu.SMEM((1,), jnp.int32),
        ],
    )(x_sp, idcol, l_table)


# speedup vs baseline: 3.7329x; 3.7329x over previous
import jax
import jax.numpy as jnp
from jax.experimental import pallas as pl
from jax.experimental.pallas import tpu as pltpu

_ST = 128
_NSUB = 10
_BLK = _ST * _NSUB
_R = 8


def _rot_down(a, k):
    return pltpu.roll(a, k % _R, 0)


def _tailmax(x_ref, id_ref, base, rows, last_id):
    D = x_ref.shape[1]
    neg = jnp.float32(-jnp.inf)
    acc = None
    for v in range(rows // _R):
        r0 = base + v * _R
        idb = jnp.broadcast_to(id_ref[pl.ds(r0, _R), :], (_R, D))
        contrib = jnp.where(idb == last_id, x_ref[pl.ds(r0, _R), :], neg)
        acc = contrib if acc is None else jnp.maximum(acc, contrib)
    return jnp.max(acc, axis=0, keepdims=True)


def _a_body(x_ref, id_ref, l_ref, c_ref, sid_ref):
    i = pl.program_id(0)
    B = x_ref.shape[0]
    neg = jnp.float32(-jnp.inf)

    first_id = id_ref[0, 0]
    last_id = id_ref[B - 1, 0]
    single = first_id == last_id

    c_id_prev = jnp.where(i == 0, -1, sid_ref[0])
    c_vec_prev = c_ref[0:1, :]
    l_row = jnp.where(first_id == c_id_prev, c_vec_prev,
                      jnp.full_like(c_vec_prev, neg))
    l_ref[0, 0:1, :] = l_row

    tm = _tailmax(x_ref, id_ref, 0, B, last_id)
    c_ref[0:1, :] = jnp.maximum(
        tm, jnp.where(single, l_row, jnp.full_like(l_row, neg)))
    sid_ref[0] = last_id


def _scan_subtile(x_ref, id_ref, out_ref, base, l_row, r_vec, r_id):
    D = x_ref.shape[1]
    nv = _ST // _R
    first_id = id_ref[base, 0]
    last_id = id_ref[base + _ST - 1, 0]
    single = first_id == last_id

    idb, f, b = [], [], []
    for v in range(nv):
        r0 = base + v * _R
        idb.append(jnp.broadcast_to(id_ref[pl.ds(r0, _R), :], (_R, D)))
        xv = x_ref[pl.ds(r0, _R), :]
        f.append(xv)
        b.append(xv)

    for k in (1, 2, 4):
        nf, nb = [], []
        for v in range(nv):
            id_d = _rot_down(idb[v], k)
            id_u = _rot_down(idb[v], -k)
            nf.append(jnp.where(idb[v] == id_d,
                                jnp.maximum(f[v], _rot_down(f[v], k)), f[v]))
            nb.append(jnp.where(idb[v] == id_u,
                                jnp.maximum(b[v], _rot_down(b[v], -k)), b[v]))
        f, b = nf, nb

    gf = [jnp.broadcast_to(f[v][_R - 1:_R, :], (_R, D)) for v in range(nv)]
    id_l = [jnp.broadcast_to(idb[v][_R - 1:_R, :], (_R, D)) for v in range(nv)]
    gb = [jnp.broadcast_to(b[v][0:1, :], (_R, D)) for v in range(nv)]
    id_f = [jnp.broadcast_to(idb[v][0:1, :], (_R, D)) for v in range(nv)]

    off = 1
    while off < nv:
        ngf, ngb = [], []
        for v in range(nv):
            d = (v - off) % nv
            u = (v + off) % nv
            ngf.append(jnp.where(id_l[v] == id_l[d],
                                 jnp.maximum(gf[v], gf[d]), gf[v]))
            ngb.append(jnp.where(id_f[v] == id_f[u],
                                 jnp.maximum(gb[v], gb[u]), gb[v]))
        gf, gb = ngf, ngb
        off *= 2

    nf, nb = [], []
    for v in range(nv):
        d = (v - 1) % nv
        u = (v + 1) % nv
        nf.append(jnp.where(idb[v] == id_l[d],
                            jnp.maximum(f[v], gf[d]), f[v]))
        nb.append(jnp.where(idb[v] == id_f[u],
                            jnp.maximum(b[v], gb[u]), b[v]))
    f, b = nf, nb

    l_bc = jnp.broadcast_to(l_row, (_R, D))
    r_bc = jnp.broadcast_to(r_vec, (_R, D))
    for v in range(nv):
        m = jnp.maximum(f[v], b[v])
        m = jnp.where(idb[v] == first_id, jnp.maximum(m, l_bc), m)
        m = jnp.where(idb[v] == r_id, jnp.maximum(m, r_bc), m)
        out_ref[pl.ds(base + v * _R, _R), :] = m

    return b[0][0:1, :], first_id, last_id, single


def _b_body(x_ref, id_ref, l_ref, out_ref, r_ref, sid_ref):
    i = pl.program_id(0)
    D = x_ref.shape[1]
    neg = jnp.float32(-jnp.inf)

    r_id_prev = jnp.where(i == 0, -1, sid_ref[0])
    r_vec = r_ref[0:1, :]

    l_sub = [l_ref[0, 0:1, :]]
    for s in range(_NSUB - 1):
        base = s * _ST
        first_id = id_ref[base, 0]
        last_id = id_ref[base + _ST - 1, 0]
        single = first_id == last_id
        tm = _tailmax(x_ref, id_ref, base, _ST, last_id)
        chain = jnp.maximum(
            tm, jnp.where(single, l_sub[s], jnp.full_like(tm, neg)))
        nxt_first = id_ref[base + _ST, 0]
        l_sub.append(jnp.where(nxt_first == last_id, chain,
                               jnp.full_like(chain, neg)))

    r_id = r_id_prev
    for s in range(_NSUB - 1, -1, -1):
        b0, first_id, last_id, single = _scan_subtile(
            x_ref, id_ref, out_ref, s * _ST, l_sub[s], r_vec, r_id)
        r_vec = jnp.maximum(
            b0, jnp.where(jnp.logical_and(single, first_id == r_id),
                          r_vec, jnp.full_like(r_vec, neg)))
        r_id = first_id

    r_ref[0:1, :] = r_vec
    sid_ref[0] = r_id


def kernel(x_sp, vertex_id):
    E, D = x_sp.shape
    nb = E // _BLK
    idcol = vertex_id.reshape(E, 1)

    l_table = pl.pallas_call(
        _a_body,
        grid=(nb,),
        in_specs=[
            pl.BlockSpec((_BLK, D), lambda i: (i, 0)),
            pl.BlockSpec((_BLK, 1), lambda i: (i, 0)),
        ],
        out_specs=pl.BlockSpec((1, 1, D), lambda i: (i, 0, 0)),
        out_shape=jax.ShapeDtypeStruct((nb, 1, D), jnp.float32),
        scratch_shapes=[
            pltpu.VMEM((1, D), jnp.float32),
            pltpu.SMEM((1,), jnp.int32),
        ],
    )(x_sp, idcol)

    return pl.pallas_call(
        _b_body,
        grid=(nb,),
        in_specs=[
            pl.BlockSpec((_BLK, D), lambda i: (nb - 1 - i, 0)),
            pl.BlockSpec((_BLK, 1), lambda i: (nb - 1 - i, 0)),
            pl.BlockSpec((1, 1, D), lambda i: (nb - 1 - i, 0, 0)),
        ],
        out_specs=pl.BlockSpec((_BLK, D), lambda i: (nb - 1 - i, 0)),
        out_shape=jax.ShapeDtypeStruct((E, D), jnp.float32),
        scratch_shapes=[
            pltpu.VMEM((1, D), jnp.float32),
            pltpu.SMEM((1,), jnp.int32),
        ],
    )(x_sp, idcol, l_table)


# speedup vs baseline: 4.1492x; 1.1115x over previous
import jax
import jax.numpy as jnp
from jax.experimental import pallas as pl
from jax.experimental.pallas import tpu as pltpu

_ST = 128
_NSUB = 20
_BLK = _ST * _NSUB
_R = 8


def _rot_down(a, k):
    return pltpu.roll(a, k % _R, 0)


def _tailmax(x_ref, id_ref, base, rows, last_id):
    D = x_ref.shape[1]
    neg = jnp.float32(-jnp.inf)
    acc = None
    for v in range(rows // _R):
        r0 = base + v * _R
        idb = jnp.broadcast_to(id_ref[pl.ds(r0, _R), :], (_R, D))
        contrib = jnp.where(idb == last_id, x_ref[pl.ds(r0, _R), :], neg)
        acc = contrib if acc is None else jnp.maximum(acc, contrib)
    return jnp.max(acc, axis=0, keepdims=True)


def _a_body(x_ref, id_ref, l_ref, c_ref, sid_ref):
    i = pl.program_id(0)
    B = x_ref.shape[0]
    neg = jnp.float32(-jnp.inf)

    first_id = id_ref[0, 0]
    last_id = id_ref[B - 1, 0]
    single = first_id == last_id

    c_id_prev = jnp.where(i == 0, -1, sid_ref[0])
    c_vec_prev = c_ref[0:1, :]
    l_row = jnp.where(first_id == c_id_prev, c_vec_prev,
                      jnp.full_like(c_vec_prev, neg))
    l_ref[0, 0:1, :] = l_row

    tm = _tailmax(x_ref, id_ref, 0, B, last_id)
    c_ref[0:1, :] = jnp.maximum(
        tm, jnp.where(single, l_row, jnp.full_like(l_row, neg)))
    sid_ref[0] = last_id


def _scan_subtile(x_ref, id_ref, out_ref, base, l_row, r_vec, r_id):
    D = x_ref.shape[1]
    nv = _ST // _R
    first_id = id_ref[base, 0]
    last_id = id_ref[base + _ST - 1, 0]
    single = first_id == last_id

    idb, f, b = [], [], []
    for v in range(nv):
        r0 = base + v * _R
        idb.append(jnp.broadcast_to(id_ref[pl.ds(r0, _R), :], (_R, D)))
        xv = x_ref[pl.ds(r0, _R), :]
        f.append(xv)
        b.append(xv)

    for k in (1, 2, 4):
        nf, nb = [], []
        for v in range(nv):
            id_d = _rot_down(idb[v], k)
            id_u = _rot_down(idb[v], -k)
            nf.append(jnp.where(idb[v] == id_d,
                                jnp.maximum(f[v], _rot_down(f[v], k)), f[v]))
            nb.append(jnp.where(idb[v] == id_u,
                                jnp.maximum(b[v], _rot_down(b[v], -k)), b[v]))
        f, b = nf, nb

    gf = [jnp.broadcast_to(f[v][_R - 1:_R, :], (_R, D)) for v in range(nv)]
    id_l = [jnp.broadcast_to(idb[v][_R - 1:_R, :], (_R, D)) for v in range(nv)]
    gb = [jnp.broadcast_to(b[v][0:1, :], (_R, D)) for v in range(nv)]
    id_f = [jnp.broadcast_to(idb[v][0:1, :], (_R, D)) for v in range(nv)]

    off = 1
    while off < nv:
        ngf, ngb = [], []
        for v in range(nv):
            d = (v - off) % nv
            u = (v + off) % nv
            ngf.append(jnp.where(id_l[v] == id_l[d],
                                 jnp.maximum(gf[v], gf[d]), gf[v]))
            ngb.append(jnp.where(id_f[v] == id_f[u],
                                 jnp.maximum(gb[v], gb[u]), gb[v]))
        gf, gb = ngf, ngb
        off *= 2

    nf, nb = [], []
    for v in range(nv):
        d = (v - 1) % nv
        u = (v + 1) % nv
        nf.append(jnp.where(idb[v] == id_l[d],
                            jnp.maximum(f[v], gf[d]), f[v]))
        nb.append(jnp.where(idb[v] == id_f[u],
                            jnp.maximum(b[v], gb[u]), b[v]))
    f, b = nf, nb

    l_bc = jnp.broadcast_to(l_row, (_R, D))
    r_bc = jnp.broadcast_to(r_vec, (_R, D))
    for v in range(nv):
        m = jnp.maximum(f[v], b[v])
        m = jnp.where(idb[v] == first_id, jnp.maximum(m, l_bc), m)
        m = jnp.where(idb[v] == r_id, jnp.maximum(m, r_bc), m)
        out_ref[pl.ds(base + v * _R, _R), :] = m

    return b[0][0:1, :], first_id, last_id, single


def _b_body(x_ref, id_ref, l_ref, out_ref, r_ref, sid_ref):
    i = pl.program_id(0)
    D = x_ref.shape[1]
    neg = jnp.float32(-jnp.inf)

    r_id_prev = jnp.where(i == 0, -1, sid_ref[0])
    r_vec = r_ref[0:1, :]

    l_sub = [l_ref[0, 0:1, :]]
    for s in range(_NSUB - 1):
        base = s * _ST
        first_id = id_ref[base, 0]
        last_id = id_ref[base + _ST - 1, 0]
        single = first_id == last_id
        tm = _tailmax(x_ref, id_ref, base, _ST, last_id)
        chain = jnp.maximum(
            tm, jnp.where(single, l_sub[s], jnp.full_like(tm, neg)))
        nxt_first = id_ref[base + _ST, 0]
        l_sub.append(jnp.where(nxt_first == last_id, chain,
                               jnp.full_like(chain, neg)))

    r_id = r_id_prev
    for s in range(_NSUB - 1, -1, -1):
        b0, first_id, last_id, single = _scan_subtile(
            x_ref, id_ref, out_ref, s * _ST, l_sub[s], r_vec, r_id)
        r_vec = jnp.maximum(
            b0, jnp.where(jnp.logical_and(single, first_id == r_id),
                          r_vec, jnp.full_like(r_vec, neg)))
        r_id = first_id

    r_ref[0:1, :] = r_vec
    sid_ref[0] = r_id


def kernel(x_sp, vertex_id):
    E, D = x_sp.shape
    nb = E // _BLK
    idcol = vertex_id.reshape(E, 1)

    l_table = pl.pallas_call(
        _a_body,
        grid=(nb,),
        in_specs=[
            pl.BlockSpec((_BLK, D), lambda i: (i, 0)),
            pl.BlockSpec((_BLK, 1), lambda i: (i, 0)),
        ],
        out_specs=pl.BlockSpec((1, 1, D), lambda i: (i, 0, 0)),
        out_shape=jax.ShapeDtypeStruct((nb, 1, D), jnp.float32),
        scratch_shapes=[
            pltpu.VMEM((1, D), jnp.float32),
            pltpu.SMEM((1,), jnp.int32),
        ],
    )(x_sp, idcol)

    return pl.pallas_call(
        _b_body,
        grid=(nb,),
        in_specs=[
            pl.BlockSpec((_BLK, D), lambda i: (nb - 1 - i, 0)),
            pl.BlockSpec((_BLK, 1), lambda i: (nb - 1 - i, 0)),
            pl.BlockSpec((1, 1, D), lambda i: (nb - 1 - i, 0, 0)),
        ],
        out_specs=pl.BlockSpec((_BLK, D), lambda i: (nb - 1 - i, 0)),
        out_shape=jax.ShapeDtypeStruct((E, D), jnp.float32),
        scratch_shapes=[
            pltpu.VMEM((1, D), jnp.float32),
            pltpu.SMEM((1,), jnp.int32),
        ],
    )(x_sp, idcol, l_table)


# speedup vs baseline: 4.2501x; 1.0243x over previous
import jax
import jax.numpy as jnp
from jax.experimental import pallas as pl
from jax.experimental.pallas import tpu as pltpu

_ST = 128
_NSUB = 25
_BLK = _ST * _NSUB
_R = 8


def _rot_down(a, k):
    return pltpu.roll(a, k % _R, 0)


def _tailmax(x_ref, id_ref, base, rows, last_id):
    D = x_ref.shape[1]
    neg = jnp.float32(-jnp.inf)
    acc = None
    for v in range(rows // _R):
        r0 = base + v * _R
        idb = jnp.broadcast_to(id_ref[pl.ds(r0, _R), :], (_R, D))
        contrib = jnp.where(idb == last_id, x_ref[pl.ds(r0, _R), :], neg)
        acc = contrib if acc is None else jnp.maximum(acc, contrib)
    return jnp.max(acc, axis=0, keepdims=True)


def _a_body(x_ref, id_ref, l_ref, c_ref, sid_ref):
    i = pl.program_id(0)
    B = x_ref.shape[0]
    neg = jnp.float32(-jnp.inf)

    first_id = id_ref[0, 0]
    last_id = id_ref[B - 1, 0]
    single = first_id == last_id

    c_id_prev = jnp.where(i == 0, -1, sid_ref[0])
    c_vec_prev = c_ref[0:1, :]
    l_row = jnp.where(first_id == c_id_prev, c_vec_prev,
                      jnp.full_like(c_vec_prev, neg))
    l_ref[0, 0:1, :] = l_row

    tm = _tailmax(x_ref, id_ref, 0, B, last_id)
    c_ref[0:1, :] = jnp.maximum(
        tm, jnp.where(single, l_row, jnp.full_like(l_row, neg)))
    sid_ref[0] = last_id


def _scan_subtile(x_ref, id_ref, out_ref, base, l_row, r_vec, r_id):
    D = x_ref.shape[1]
    nv = _ST // _R
    first_id = id_ref[base, 0]
    last_id = id_ref[base + _ST - 1, 0]
    single = first_id == last_id

    idb, f, b = [], [], []
    for v in range(nv):
        r0 = base + v * _R
        idb.append(jnp.broadcast_to(id_ref[pl.ds(r0, _R), :], (_R, D)))
        xv = x_ref[pl.ds(r0, _R), :]
        f.append(xv)
        b.append(xv)

    for k in (1, 2, 4):
        nf, nb = [], []
        for v in range(nv):
            id_d = _rot_down(idb[v], k)
            id_u = _rot_down(idb[v], -k)
            nf.append(jnp.where(idb[v] == id_d,
                                jnp.maximum(f[v], _rot_down(f[v], k)), f[v]))
            nb.append(jnp.where(idb[v] == id_u,
                                jnp.maximum(b[v], _rot_down(b[v], -k)), b[v]))
        f, b = nf, nb

    gf = [jnp.broadcast_to(f[v][_R - 1:_R, :], (_R, D)) for v in range(nv)]
    id_l = [jnp.broadcast_to(idb[v][_R - 1:_R, :], (_R, D)) for v in range(nv)]
    gb = [jnp.broadcast_to(b[v][0:1, :], (_R, D)) for v in range(nv)]
    id_f = [jnp.broadcast_to(idb[v][0:1, :], (_R, D)) for v in range(nv)]

    off = 1
    while off < nv:
        ngf, ngb = [], []
        for v in range(nv):
            d = (v - off) % nv
            u = (v + off) % nv
            ngf.append(jnp.where(id_l[v] == id_l[d],
                                 jnp.maximum(gf[v], gf[d]), gf[v]))
            ngb.append(jnp.where(id_f[v] == id_f[u],
                                 jnp.maximum(gb[v], gb[u]), gb[v]))
        gf, gb = ngf, ngb
        off *= 2

    nf, nb = [], []
    for v in range(nv):
        d = (v - 1) % nv
        u = (v + 1) % nv
        nf.append(jnp.where(idb[v] == id_l[d],
                            jnp.maximum(f[v], gf[d]), f[v]))
        nb.append(jnp.where(idb[v] == id_f[u],
                            jnp.maximum(b[v], gb[u]), b[v]))
    f, b = nf, nb

    l_bc = jnp.broadcast_to(l_row, (_R, D))
    r_bc = jnp.broadcast_to(r_vec, (_R, D))
    for v in range(nv):
        m = jnp.maximum(f[v], b[v])
        m = jnp.where(idb[v] == first_id, jnp.maximum(m, l_bc), m)
        m = jnp.where(idb[v] == r_id, jnp.maximum(m, r_bc), m)
        out_ref[pl.ds(base + v * _R, _R), :] = m

    return b[0][0:1, :], first_id, last_id, single


def _b_body(x_ref, id_ref, l_ref, out_ref, r_ref, sid_ref):
    i = pl.program_id(0)
    D = x_ref.shape[1]
    neg = jnp.float32(-jnp.inf)

    r_id_prev = jnp.where(i == 0, -1, sid_ref[0])
    r_vec = r_ref[0:1, :]

    l_sub = [l_ref[0, 0:1, :]]
    for s in range(_NSUB - 1):
        base = s * _ST
        first_id = id_ref[base, 0]
        last_id = id_ref[base + _ST - 1, 0]
        single = first_id == last_id
        tm = _tailmax(x_ref, id_ref, base, _ST, last_id)
        chain = jnp.maximum(
            tm, jnp.where(single, l_sub[s], jnp.full_like(tm, neg)))
        nxt_first = id_ref[base + _ST, 0]
        l_sub.append(jnp.where(nxt_first == last_id, chain,
                               jnp.full_like(chain, neg)))

    r_id = r_id_prev
    for s in range(_NSUB - 1, -1, -1):
        b0, first_id, last_id, single = _scan_subtile(
            x_ref, id_ref, out_ref, s * _ST, l_sub[s], r_vec, r_id)
        r_vec = jnp.maximum(
            b0, jnp.where(jnp.logical_and(single, first_id == r_id),
                          r_vec, jnp.full_like(r_vec, neg)))
        r_id = first_id

    r_ref[0:1, :] = r_vec
    sid_ref[0] = r_id


def kernel(x_sp, vertex_id):
    E, D = x_sp.shape
    nb = E // _BLK
    idcol = vertex_id.reshape(E, 1)

    l_table = pl.pallas_call(
        _a_body,
        grid=(nb,),
        in_specs=[
            pl.BlockSpec((_BLK, D), lambda i: (i, 0)),
            pl.BlockSpec((_BLK, 1), lambda i: (i, 0)),
        ],
        out_specs=pl.BlockSpec((1, 1, D), lambda i: (i, 0, 0)),
        out_shape=jax.ShapeDtypeStruct((nb, 1, D), jnp.float32),
        scratch_shapes=[
            pltpu.VMEM((1, D), jnp.float32),
            pltpu.SMEM((1,), jnp.int32),
        ],
    )(x_sp, idcol)

    return pl.pallas_call(
        _b_body,
        grid=(nb,),
        in_specs=[
            pl.BlockSpec((_BLK, D), lambda i: (nb - 1 - i, 0)),
            pl.BlockSpec((_BLK, 1), lambda i: (nb - 1 - i, 0)),
            pl.BlockSpec((1, 1, D), lambda i: (nb - 1 - i, 0, 0)),
        ],
        out_specs=pl.BlockSpec((_BLK, D), lambda i: (nb - 1 - i, 0)),
        out_shape=jax.ShapeDtypeStruct((E, D), jnp.float32),
        scratch_shapes=[
            pltpu.VMEM((1, D), jnp.float32),
            pltpu.SMEM((1,), jnp.int32),
        ],
    )(x_sp, idcol, l_table)
